# Initial kernel scaffold; baseline (speedup 1.0000x reference)
import jax
import jax.numpy as jnp
from jax.experimental import pallas as pl


def kernel(supports, queries, support_labels, W):
    raise NotImplementedError("write your pallas kernel here")


# speedup vs baseline: 3.1169x; 3.1169x over previous
import jax
import jax.numpy as jnp
from jax.experimental import pallas as pl

_N_CLS = 5
_K = 10
_Q = 10
_EPB = 8


def _sim_block(s_ref, q_ref, lab_ref, w_ref, out_ref):
    E = _EPB
    R = E * _N_CLS * _K
    S = E * _N_CLS
    W = w_ref[:]

    emb_s = jnp.dot(s_ref[:], W, preferred_element_type=jnp.float32)
    emb_q = jnp.dot(q_ref[:], W, preferred_element_type=jnp.float32)

    lab = lab_ref[0]
    ep = jax.lax.broadcasted_iota(jnp.int32, (1, R), 1) // (_N_CLS * _K)
    seg = ep * _N_CLS + lab
    seg_ids = jax.lax.broadcasted_iota(jnp.int32, (S, 1), 0)
    onehot_t = (seg_ids == seg).astype(jnp.float32)

    counts = jnp.sum(onehot_t, axis=1, keepdims=True)
    sums = jnp.dot(onehot_t, emb_s, preferred_element_type=jnp.float32)
    proto = sums / jnp.maximum(counts, 1.0)

    qn = jnp.sqrt(jnp.sum(emb_q * emb_q, axis=1, keepdims=True))
    pn2 = jax.lax.dot_general(
        jnp.ones((1, W.shape[0]), jnp.float32), proto * proto,
        (((1,), (1,)), ((), ())), preferred_element_type=jnp.float32)
    pn = jnp.sqrt(pn2)

    num = jax.lax.dot_general(
        emb_q, proto, (((1,), (1,)), ((), ())),
        preferred_element_type=jnp.float32)
    sims = num / (qn * pn + 1e-8)

    row_ep = jax.lax.broadcasted_iota(jnp.int32, (R, S), 0) // (_N_CLS * _K)
    col_ep = jax.lax.broadcasted_iota(jnp.int32, (R, S), 1) // _N_CLS
    masked = sims * (row_ep == col_ep).astype(jnp.float32)
    sel = (jax.lax.broadcasted_iota(jnp.int32, (S, _N_CLS), 0) % _N_CLS
           == jax.lax.broadcasted_iota(jnp.int32, (S, _N_CLS), 1)
           ).astype(jnp.float32)
    out_ref[:] = jnp.dot(masked, sel, preferred_element_type=jnp.float32)


def kernel(supports, queries, support_labels, W):
    n_rows, D = supports.shape
    n_seg_rows = _N_CLS * _K
    B = n_rows // n_seg_rows
    E = _EPB
    n_blocks = B // E
    R = E * n_seg_rows

    labels3 = support_labels.reshape(n_blocks, 1, R)

    out = pl.pallas_call(
        _sim_block,
        grid=(n_blocks,),
        in_specs=[
            pl.BlockSpec((R, D), lambda i: (i, 0)),
            pl.BlockSpec((R, D), lambda i: (i, 0)),
            pl.BlockSpec((1, 1, R), lambda i: (i, 0, 0)),
            pl.BlockSpec((D, D), lambda i: (0, 0)),
        ],
        out_specs=pl.BlockSpec((R, _N_CLS), lambda i: (i, 0)),
        out_shape=jax.ShapeDtypeStruct((n_rows, _N_CLS), jnp.float32),
    )(supports, queries, labels3, W)

    return out.reshape(-1)


# speedup vs baseline: 4.8338x; 1.5509x over previous
import jax
import jax.numpy as jnp
from jax.experimental import pallas as pl

_N_CLS = 5
_K = 10
_Q = 10
_EPB = 16


def _sim_block(s_ref, q_ref, lab_ref, w_ref, out_ref):
    E = _EPB
    R = E * _N_CLS * _K
    S = E * _N_CLS
    W = w_ref[:]

    lab = lab_ref[0]
    ep = jax.lax.broadcasted_iota(jnp.int32, (1, R), 1) // (_N_CLS * _K)
    seg = ep * _N_CLS + lab
    seg_ids = jax.lax.broadcasted_iota(jnp.int32, (S, 1), 0)
    onehot_t = (seg_ids == seg).astype(jnp.float32)

    counts = jnp.sum(onehot_t, axis=1, keepdims=True)
    sums = jnp.dot(onehot_t, s_ref[:], preferred_element_type=jnp.float32)
    mean_s = sums / jnp.maximum(counts, 1.0)
    proto = jnp.dot(mean_s, W, preferred_element_type=jnp.float32)

    emb_q = jnp.dot(q_ref[:], W, preferred_element_type=jnp.float32)

    qn = jnp.sqrt(jnp.sum(emb_q * emb_q, axis=1, keepdims=True))
    pn2 = jax.lax.dot_general(
        jnp.ones((1, W.shape[0]), jnp.float32), proto * proto,
        (((1,), (1,)), ((), ())), preferred_element_type=jnp.float32)
    pn = jnp.sqrt(pn2)

    num = jax.lax.dot_general(
        emb_q, proto, (((1,), (1,)), ((), ())),
        preferred_element_type=jnp.float32)
    sims = num / (qn * pn + 1e-8)

    row_ep = jax.lax.broadcasted_iota(jnp.int32, (R, S), 0) // (_N_CLS * _Q)
    col_ep = jax.lax.broadcasted_iota(jnp.int32, (R, S), 1) // _N_CLS
    masked = sims * (row_ep == col_ep).astype(jnp.float32)
    sel = (jax.lax.broadcasted_iota(jnp.int32, (S, _N_CLS), 0) % _N_CLS
           == jax.lax.broadcasted_iota(jnp.int32, (S, _N_CLS), 1)
           ).astype(jnp.float32)
    out_ref[:] = jnp.dot(masked, sel, preferred_element_type=jnp.float32)


def kernel(supports, queries, support_labels, W):
    n_rows, D = supports.shape
    n_seg_rows = _N_CLS * _K
    B = n_rows // n_seg_rows
    E = _EPB
    n_blocks = B // E
    R = E * n_seg_rows

    labels3 = support_labels.reshape(n_blocks, 1, R)

    out = pl.pallas_call(
        _sim_block,
        grid=(n_blocks,),
        in_specs=[
            pl.BlockSpec((R, D), lambda i: (i, 0)),
            pl.BlockSpec((R, D), lambda i: (i, 0)),
            pl.BlockSpec((1, 1, R), lambda i: (i, 0, 0)),
            pl.BlockSpec((D, D), lambda i: (0, 0)),
        ],
        out_specs=pl.BlockSpec((R, _N_CLS), lambda i: (i, 0)),
        out_shape=jax.ShapeDtypeStruct((n_rows, _N_CLS), jnp.float32),
    )(supports, queries, labels3, W)

    return out.reshape(-1)
